# Initial kernel scaffold; baseline (speedup 1.0000x reference)
import jax
import jax.numpy as jnp
from jax.experimental import pallas as pl


def kernel(queries, keys):
    raise NotImplementedError("write your pallas kernel here")


# speedup vs baseline: 1.0255x; 1.0255x over previous
import functools

import jax
import jax.numpy as jnp
from jax.experimental import pallas as pl
from jax.experimental.pallas import tpu as pltpu


def _ball_body(K, radius, q_ref, kx_ref, ky_ref, dist_ref, idx_ref, d2_ref):
    B = q_ref.shape[0]
    N = kx_ref.shape[1]
    q = q_ref[...]
    qx = q[:, 0:1]
    qy = q[:, 1:2]
    kx = kx_ref[...]
    ky = ky_ref[...]
    q2 = qx * qx + qy * qy
    k2 = kx * kx + ky * ky
    qxb = qx.astype(jnp.bfloat16).astype(jnp.float32)
    qyb = qy.astype(jnp.bfloat16).astype(jnp.float32)
    kxb = kx.astype(jnp.bfloat16).astype(jnp.float32)
    kyb = ky.astype(jnp.bfloat16).astype(jnp.float32)
    dot = qxb * kxb + qyb * kyb
    d2 = (q2 + k2) - 2.0 * dot
    d2 = jnp.maximum(d2, 0.0)
    r2 = radius * radius
    inf = jnp.float32(jnp.inf)
    d2_ref[...] = jnp.where(d2 <= r2, d2, inf)

    iota = jax.lax.broadcasted_iota(jnp.int32, (1, N), 1)
    kiota = jax.lax.broadcasted_iota(jnp.int32, (B, K), 1)
    big = jnp.int32(2**31 - 1)

    def step(t, carry):
        dist_acc, idx_acc = carry
        d2c = d2_ref[...]
        m = jnp.min(d2c, axis=1, keepdims=True)
        pos = jnp.min(jnp.where(d2c == m, iota, big), axis=1, keepdims=True)
        valid = m <= r2
        mv = jnp.where(valid, m, 0.0)
        pv = jnp.where(valid, pos, -1)
        sel = kiota == t
        dist_acc = jnp.where(sel, mv, dist_acc)
        idx_acc = jnp.where(sel, pv, idx_acc)
        d2_ref[...] = jnp.where(iota == pos, inf, d2c)
        return dist_acc, idx_acc

    dist_acc = jnp.zeros((B, K), jnp.float32)
    idx_acc = jnp.zeros((B, K), jnp.int32)
    dist_acc, idx_acc = jax.lax.fori_loop(0, K, step, (dist_acc, idx_acc))
    dist_ref[...] = dist_acc
    idx_ref[...] = idx_acc


def _ball_query(q, kx, ky, K, radius, q_block):
    Q = q.shape[0]
    N = kx.shape[1]
    grid = (Q // q_block,)
    body = functools.partial(_ball_body, K, radius)
    return pl.pallas_call(
        body,
        grid=grid,
        in_specs=[
            pl.BlockSpec((q_block, 2), lambda i: (i, 0)),
            pl.BlockSpec((1, N), lambda i: (0, 0)),
            pl.BlockSpec((1, N), lambda i: (0, 0)),
        ],
        out_specs=[
            pl.BlockSpec((q_block, K), lambda i: (i, 0)),
            pl.BlockSpec((q_block, K), lambda i: (i, 0)),
        ],
        out_shape=[
            jax.ShapeDtypeStruct((Q, K), jnp.float32),
            jax.ShapeDtypeStruct((Q, K), jnp.int32),
        ],
        scratch_shapes=[pltpu.VMEM((q_block, N), jnp.float32)],
    )(q, kx, ky)


def kernel(queries, keys):
    kx = keys[:, 0].reshape(1, -1)
    ky = keys[:, 1].reshape(1, -1)
    qx = queries[:, 0].reshape(1, -1)
    qy = queries[:, 1].reshape(1, -1)
    dists_dst, idx_dst = _ball_query(queries, kx, ky, 64, 34.0, 8)
    dists_src, idx_src = _ball_query(queries, qx, qy, 8, 10.0, 64)
    return dists_dst, idx_dst, dists_src, idx_src
